# Initial kernel scaffold; baseline (speedup 1.0000x reference)
import jax
import jax.numpy as jnp
from jax.experimental import pallas as pl


def kernel(z, edge_index):
    raise NotImplementedError("write your pallas kernel here")


# speedup vs baseline: 1.2296x; 1.2296x over previous
import functools

import jax
import jax.numpy as jnp
from jax import lax
from jax.experimental import pallas as pl
from jax.experimental.pallas import tpu as pltpu
from jax.experimental.pallas import tpu_sc as plsc

N_NODES = 10000
D = 128
E = 320000
L = 16
NW = 32
EPW = E // NW
CH = 80
NCHUNKS = EPW // CH


def _normalize_body(z_ref, o_ref):
    x = z_ref[...]
    ss = jnp.sum(x * x, axis=1, keepdims=True)
    o_ref[...] = x * lax.rsqrt(jnp.maximum(ss, 1e-12))


def _normalize(z):
    n = z.shape[0]
    blk = 2000
    return pl.pallas_call(
        _normalize_body,
        grid=(n // blk,),
        in_specs=[pl.BlockSpec((blk, D), lambda i: (i, 0))],
        out_specs=pl.BlockSpec((blk, D), lambda i: (i, 0)),
        out_shape=jax.ShapeDtypeStruct((n, D), jnp.float32),
    )(z)


def _edge_kernel(zn, srci, dsti, out, sv, dv, arows, brows, outv, sema, semb):
    wid = lax.axis_index("s") * 2 + lax.axis_index("c")
    base = pl.multiple_of(wid * EPW, 8)

    pltpu.sync_copy(srci.at[pl.ds(base, EPW)], sv)
    pltpu.sync_copy(dsti.at[pl.ds(base, EPW)], dv)

    def chunk_body(g, carry):
        coff = pl.multiple_of(g * CH, 8)
        a_cp = pltpu.async_copy(zn.at[sv.at[pl.ds(coff, CH)]], arows, sema)
        b_cp = pltpu.async_copy(zn.at[dv.at[pl.ds(coff, CH)]], brows, semb)
        a_cp.wait()
        b_cp.wait()

        for e0 in range(0, CH, L):
            ev = e0 + lax.iota(jnp.int32, L)
            acc = jnp.zeros((L,), jnp.float32)
            for d in range(D):
                dcol = jnp.full((L,), d, jnp.int32)
                av = plsc.load_gather(arows, [ev, dcol])
                bv = plsc.load_gather(brows, [ev, dcol])
                acc = acc + av * bv
            outv[pl.ds(e0, L)] = 1.0 / (1.0 + jnp.exp(-acc))

        oof = pl.multiple_of(base + g * CH, 8)
        pltpu.sync_copy(outv, out.at[pl.ds(oof, CH)])
        return carry

    lax.fori_loop(0, NCHUNKS, chunk_body, 0)


def _make_sc_call():
    mesh = plsc.VectorSubcoreMesh(core_axis_name="c", subcore_axis_name="s")
    return functools.partial(
        pl.kernel,
        mesh=mesh,
        compiler_params=pltpu.CompilerParams(needs_layout_passes=False),
        out_type=jax.ShapeDtypeStruct((E,), jnp.float32),
        scratch_types=[
            pltpu.VMEM((EPW,), jnp.int32),
            pltpu.VMEM((EPW,), jnp.int32),
            pltpu.VMEM((CH, D), jnp.float32),
            pltpu.VMEM((CH, D), jnp.float32),
            pltpu.VMEM((CH,), jnp.float32),
            pltpu.SemaphoreType.DMA,
            pltpu.SemaphoreType.DMA,
        ],
    )(_edge_kernel)


def kernel(z, edge_index):
    zn = _normalize(z)
    src = edge_index[0]
    dst = edge_index[1]
    return _make_sc_call()(zn, src, dst)


# speedup vs baseline: 5.2336x; 4.2564x over previous
import functools

import jax
import jax.numpy as jnp
from jax import lax
from jax.experimental import pallas as pl
from jax.experimental.pallas import tpu as pltpu
from jax.experimental.pallas import tpu_sc as plsc

N_NODES = 10000
D = 128
E = 320000
L = 16
NW = 32
EPW = E // NW
CH = 80
NCHUNKS = EPW // CH


def _normalize_body(z_ref, o_ref):
    x = z_ref[...]
    ss = jnp.sum(x * x, axis=1, keepdims=True)
    o_ref[...] = x * lax.rsqrt(jnp.maximum(ss, 1e-12))


def _normalize(z):
    n = z.shape[0]
    blk = 2000
    return pl.pallas_call(
        _normalize_body,
        grid=(n // blk,),
        in_specs=[pl.BlockSpec((blk, D), lambda i: (i, 0))],
        out_specs=pl.BlockSpec((blk, D), lambda i: (i, 0)),
        out_shape=jax.ShapeDtypeStruct((n, D), jnp.float32),
    )(z)


def _edge_kernel(zn, srci, dsti, out, sv, dv, arows, brows, outv, sema, semb):
    wid = lax.axis_index("s") * 2 + lax.axis_index("c")
    base = pl.multiple_of(wid * EPW, 8)

    pltpu.sync_copy(srci.at[pl.ds(base, EPW)], sv)
    pltpu.sync_copy(dsti.at[pl.ds(base, EPW)], dv)

    def chunk_body(g, carry):
        coff = pl.multiple_of(g * CH, 8)
        a_cp = pltpu.async_copy(zn.at[sv.at[pl.ds(coff, CH)]], arows, sema)
        b_cp = pltpu.async_copy(zn.at[dv.at[pl.ds(coff, CH)]], brows, semb)
        a_cp.wait()
        b_cp.wait()

        for e0 in range(0, CH, L):
            ev = e0 + lax.iota(jnp.int32, L)
            dv_ = lax.iota(jnp.int32, L)
            acc = jnp.zeros((L,), jnp.float32)
            for _t in range(D):
                av = plsc.load_gather(arows, [ev, dv_])
                bv = plsc.load_gather(brows, [ev, dv_])
                acc = acc + av * bv
                dv_ = (dv_ + 1) & (D - 1)
            outv[pl.ds(e0, L)] = 1.0 / (1.0 + jnp.exp(-acc))

        oof = pl.multiple_of(base + g * CH, 8)
        pltpu.sync_copy(outv, out.at[pl.ds(oof, CH)])
        return carry

    lax.fori_loop(0, NCHUNKS, chunk_body, 0)


def _make_sc_call():
    mesh = plsc.VectorSubcoreMesh(core_axis_name="c", subcore_axis_name="s")
    return functools.partial(
        pl.kernel,
        mesh=mesh,
        compiler_params=pltpu.CompilerParams(needs_layout_passes=False),
        out_type=jax.ShapeDtypeStruct((E,), jnp.float32),
        scratch_types=[
            pltpu.VMEM((EPW,), jnp.int32),
            pltpu.VMEM((EPW,), jnp.int32),
            pltpu.VMEM((CH, D), jnp.float32),
            pltpu.VMEM((CH, D), jnp.float32),
            pltpu.VMEM((CH,), jnp.float32),
            pltpu.SemaphoreType.DMA,
            pltpu.SemaphoreType.DMA,
        ],
    )(_edge_kernel)


def kernel(z, edge_index):
    zn = _normalize(z)
    src = edge_index[0]
    dst = edge_index[1]
    return _make_sc_call()(zn, src, dst)
